# baseline (speedup 1.0000x reference)
import jax
import jax.numpy as jnp
from jax.experimental import pallas as pl


_ROWS_PER_BLOCK = 512


def _gate_kernel(x_ref, wt_ref, out_ref):
    out_ref[...] = jnp.dot(x_ref[...], wt_ref[...],
                           preferred_element_type=jnp.float32)


def kernel(hidden_states, weight):
    bsz, seq_len, h = hidden_states.shape
    n_exp = weight.shape[0]
    rows = bsz * seq_len
    x = hidden_states.reshape(rows, h)
    wt = weight.T

    grid = (rows // _ROWS_PER_BLOCK,)
    out = pl.pallas_call(
        _gate_kernel,
        grid=grid,
        in_specs=[
            pl.BlockSpec((_ROWS_PER_BLOCK, h), lambda i: (i, 0)),
            pl.BlockSpec((h, n_exp), lambda i: (0, 0)),
        ],
        out_specs=pl.BlockSpec((_ROWS_PER_BLOCK, n_exp), lambda i: (i, 0)),
        out_shape=jax.ShapeDtypeStruct((rows, n_exp), jnp.float32),
    )(x, wt)
    return out


# speedup vs baseline: 1.1163x; 1.1163x over previous
import jax
import jax.numpy as jnp
from jax.experimental import pallas as pl


_ROWS_PER_BLOCK = 2048


def _gate_kernel(x_ref, wt_ref, out_ref):
    out_ref[...] = jnp.dot(x_ref[...], wt_ref[...],
                           preferred_element_type=jnp.float32)


def kernel(hidden_states, weight):
    bsz, seq_len, h = hidden_states.shape
    n_exp = weight.shape[0]
    rows = bsz * seq_len
    x = hidden_states.reshape(rows, h)
    wt = weight.T

    grid = (rows // _ROWS_PER_BLOCK,)
    out = pl.pallas_call(
        _gate_kernel,
        grid=grid,
        in_specs=[
            pl.BlockSpec((_ROWS_PER_BLOCK, h), lambda i: (i, 0)),
            pl.BlockSpec((h, n_exp), lambda i: (0, 0)),
        ],
        out_specs=pl.BlockSpec((_ROWS_PER_BLOCK, n_exp), lambda i: (i, 0)),
        out_shape=jax.ShapeDtypeStruct((rows, n_exp), jnp.float32),
    )(x, wt)
    return out


# speedup vs baseline: 1.1592x; 1.0384x over previous
import jax
import jax.numpy as jnp
from jax.experimental import pallas as pl


_ROWS_PER_BLOCK = 1024


def _gate_kernel(x_ref, wt_ref, out_ref):
    out_ref[...] = jnp.dot(x_ref[...], wt_ref[...],
                           preferred_element_type=jnp.float32)


def kernel(hidden_states, weight):
    bsz, seq_len, h = hidden_states.shape
    n_exp = weight.shape[0]
    rows = bsz * seq_len
    x = hidden_states.reshape(rows, h)
    wt = weight.T

    grid = (rows // _ROWS_PER_BLOCK,)
    out = pl.pallas_call(
        _gate_kernel,
        grid=grid,
        in_specs=[
            pl.BlockSpec((_ROWS_PER_BLOCK, h), lambda i: (i, 0)),
            pl.BlockSpec((h, n_exp), lambda i: (0, 0)),
        ],
        out_specs=pl.BlockSpec((_ROWS_PER_BLOCK, n_exp), lambda i: (i, 0)),
        out_shape=jax.ShapeDtypeStruct((rows, n_exp), jnp.float32),
    )(x, wt)
    return out
